# baseline (speedup 1.0000x reference)
import functools

import jax
import jax.numpy as jnp
from jax import lax
from jax.experimental import pallas as pl
from jax.experimental.pallas import tpu as pltpu, tpu_sc as plsc

VOC = 1000000
EMB = 64
NB = 4096
NL = 200
NTOT = NB * NL
NIH = VOC // 128
NROW = VOC // 2
NTC = (NB // 128) * NL
OUTROWS = NTOT * EMB // 128

_params = pltpu.CompilerParams(
    use_tc_tiling_on_sc=True, needs_layout_passes=False
)


def _iota16():
    return lax.iota(jnp.int32, 16)


def _transpose_block(src, dst, n_rows, pair_base):
    lanes = _iota16()

    def row(r, carry):
        for c16 in range(8):
            e = (c16 * 16) % 64 + lanes
            j = jnp.broadcast_to(pair_base + 2 * r + (c16 // 4), (16,))
            g = plsc.load_gather(src, [e, j])
            dst[r, pl.ds(c16 * 16, 16)] = g
        return carry

    lax.fori_loop(0, n_rows, row, 0)


@functools.cache
def _build_a(nc):
    mesh = plsc.VectorSubcoreMesh(core_axis_name="c", subcore_axis_name="s")
    n_main = 244

    @functools.partial(
        pl.kernel,
        out_type=jax.ShapeDtypeStruct((NROW, 128), jnp.float32),
        mesh=mesh,
        scratch_types=[
            pltpu.VMEM((2, 64, 128), jnp.float32),
            pltpu.VMEM((2, 64, 128), jnp.float32),
            [pltpu.SemaphoreType.DMA] * 2,
            [pltpu.SemaphoreType.DMA] * 2,
        ],
        compiler_params=_params,
    )
    def ka(tableT, tailsc, scratch, tbuf, svout, fsems, wsems):
        w = lax.axis_index("s") * nc + lax.axis_index("c")

        def fire_fetch(k, b):
            ih = w + 32 * k
            for eh in range(8):
                pltpu.async_copy(
                    tableT.at[pl.ds(eh * 8, 8), pl.ds(ih * 128, 128)],
                    tbuf.at[b, pl.ds(eh * 8, 8)],
                    fsems[b],
                )

        def wait_fetch(b):
            for eh in range(8):
                pltpu.make_async_copy(
                    tableT.at[pl.ds(0, 8), pl.ds(0, 128)],
                    tbuf.at[b, pl.ds(eh * 8, 8)],
                    fsems[b],
                ).wait()

        def store(k, b):
            ih = w + 32 * k
            pltpu.async_copy(svout.at[b], scratch.at[pl.ds(ih * 64, 64)], wsems[b])

        def wait_store(b):
            pltpu.make_async_copy(
                svout.at[b], scratch.at[pl.ds(0, 64)], wsems[b]
            ).wait()

        fire_fetch(0, 0)
        fire_fetch(1, 1)

        def body(k, b, fetch_ahead, wait_wb):
            wait_fetch(b)
            if wait_wb:
                wait_store(b)
            _transpose_block(tbuf.at[b], svout.at[b], 64, 0)
            store(k, b)
            if fetch_ahead:
                fire_fetch(k + 2, b)

        def pair(i, carry):
            k = 2 + 2 * i
            body(k, 0, True, True)
            body(k + 1, 1, True, True)
            return carry

        body(0, 0, True, False)
        body(1, 1, True, False)
        lax.fori_loop(0, (n_main - 4) // 2, pair, 0)
        body(n_main - 2, 0, False, True)
        body(n_main - 1, 1, False, True)
        wait_store(0)
        wait_store(1)

        @pl.when(w >= 28)
        def _():
            ih = 7808 + (w - 28)
            for eh in range(8):
                pltpu.sync_copy(
                    tableT.at[pl.ds(eh * 8, 8), pl.ds(ih * 128, 128)],
                    tbuf.at[0, pl.ds(eh * 8, 8)],
                )
            _transpose_block(tbuf.at[0], svout.at[0], 64, 0)
            pltpu.sync_copy(svout.at[0], scratch.at[pl.ds(ih * 64, 64)])

        @pl.when(w == 27)
        def _():
            pltpu.sync_copy(tailsc, tbuf.at[0, pl.ds(0, 32)])
            pltpu.sync_copy(
                tbuf.at[0, pl.ds(0, 32)], scratch.at[pl.ds(NROW - 32, 32)]
            )

    return ka


@functools.cache
def _build_b(nc):
    mesh = plsc.VectorSubcoreMesh(core_axis_name="c", subcore_axis_name="s")
    n_tc = NTC // 32

    @functools.partial(
        pl.kernel,
        out_type=jax.ShapeDtypeStruct((OUTROWS, 128), jnp.float32),
        mesh=mesh,
        scratch_types=[
            pltpu.VMEM((2, 128), jnp.int32),
            pltpu.VMEM((2, 128), jnp.int32),
            pltpu.VMEM((2, 128), jnp.int32),
            pltpu.VMEM((2, 128, 128), jnp.float32),
            pltpu.VMEM((2, 64, 128), jnp.float32),
            [pltpu.SemaphoreType.DMA] * 2,
            [pltpu.SemaphoreType.DMA] * 2,
            [pltpu.SemaphoreType.DMA] * 2,
        ],
        compiler_params=_params,
    )
    def kb(xT, scratch, out2d, ivec, gvec, pvec, rows, outv, isems, gsems, osems):
        w = lax.axis_index("s") * nc + lax.axis_index("c")
        lanes = _iota16()

        def fire_idx(k, b):
            tc = w + 32 * k
            l = tc // 32
            bc = tc % 32
            pltpu.async_copy(
                xT.at[l, pl.ds(bc * 128, 128)], ivec.at[b], isems[b]
            )

        def wait_idx(b):
            pltpu.make_async_copy(
                xT.at[0, pl.ds(0, 128)], ivec.at[b], isems[b]
            ).wait()

        def prep_and_gather(b):
            for j16 in range(8):
                v = ivec[b, pl.ds(j16 * 16, 16)]
                gvec[b, pl.ds(j16 * 16, 16)] = lax.shift_right_logical(v, 1)
                pvec[b, pl.ds(j16 * 16, 16)] = lax.shift_left(
                    lax.bitwise_and(v, 1), 6
                )
            pltpu.async_copy(scratch.at[gvec.at[b]], rows.at[b], gsems[b])

        def wait_gather(b):
            pltpu.make_async_copy(
                scratch.at[gvec.at[b]], rows.at[b], gsems[b]
            ).wait()

        def transpose_tc(b):
            pvv = [pvec[b, pl.ds(j16 * 16, 16)] for j16 in range(8)]
            src = rows.at[b]
            dst = outv.at[b]

            def col(e, carry):
                for j16 in range(8):
                    g = plsc.load_gather(src, [lanes + j16 * 16, pvv[j16] + e])
                    dst[e, pl.ds(j16 * 16, 16)] = g
                return carry

            lax.fori_loop(0, 64, col, 0)

        def store_out(k, b):
            tc = w + 32 * k
            l = tc // 32
            bc = tc % 32
            row0 = l * 2048 + bc * 8
            for eh in range(8):
                pltpu.async_copy(
                    outv.at[b, pl.ds(eh * 8, 8)],
                    out2d.at[pl.ds(row0 + eh * 256, 8)],
                    osems[b],
                )

        def wait_store(b):
            for eh in range(8):
                pltpu.make_async_copy(
                    outv.at[b, pl.ds(0, 8)],
                    out2d.at[pl.ds(0, 8)],
                    osems[b],
                ).wait()

        fire_idx(0, 0)
        fire_idx(1, 1)
        wait_idx(0)
        prep_and_gather(0)

        def body(k, b, next_gather, fetch_ahead, wait_wb):
            nb = 1 - b
            wait_gather(b)
            if next_gather:
                wait_idx(nb)
                prep_and_gather(nb)
            if fetch_ahead:
                fire_idx(k + 2, b)
            if wait_wb:
                wait_store(b)
            transpose_tc(b)
            store_out(k, b)

        def pair(i, carry):
            k = 2 + 2 * i
            body(k, 0, True, True, True)
            body(k + 1, 1, True, True, True)
            return carry

        body(0, 0, True, True, False)
        body(1, 1, True, True, False)
        lax.fori_loop(0, (n_tc - 4) // 2, pair, 0)
        body(n_tc - 2, 0, True, False, True)
        body(n_tc - 1, 1, False, False, True)
        wait_store(0)
        wait_store(1)

    return kb


def kernel(x, table):
    info = plsc.get_sparse_core_info()
    nc = info.num_cores
    tableT = table.T
    tailsc = table[VOC - 64:].reshape(32, 128)
    xT = x.T
    scratch = _build_a(nc)(tableT, tailsc)
    out2d = _build_b(nc)(xT, scratch)
    out = out2d.reshape(NL, 8, NB // 128, 8, 128)
    out = out.transpose(2, 4, 0, 1, 3)
    return out.reshape(NB, NL, EMB)


# speedup vs baseline: 1.8624x; 1.8624x over previous
import functools

import jax
import jax.numpy as jnp
from jax import lax
from jax.experimental import pallas as pl
from jax.experimental.pallas import tpu as pltpu, tpu_sc as plsc

VOC = 1000000
EMB = 64
NB = 4096
NL = 200
NTOT = NB * NL
NIH = VOC // 128
NROW = VOC // 2
NTC = (NB // 128) * NL
OUTROWS = NTOT * EMB // 128

_params = pltpu.CompilerParams(
    use_tc_tiling_on_sc=True, needs_layout_passes=False
)


def _iota16():
    return lax.iota(jnp.int32, 16)


def _transpose_block(src, dst, n_rows, pair_base):
    lanes = _iota16()

    @plsc.parallel_loop(0, n_rows, unroll=8)
    def row(r):
        for c16 in range(8):
            e = (c16 * 16) % 64 + lanes
            j = jnp.broadcast_to(pair_base + 2 * r + (c16 // 4), (16,))
            dst[r, pl.ds(c16 * 16, 16)] = plsc.load_gather(src, [e, j])


@functools.cache
def _build_a(nc):
    mesh = plsc.VectorSubcoreMesh(core_axis_name="c", subcore_axis_name="s")
    n_main = 244

    @functools.partial(
        pl.kernel,
        out_type=jax.ShapeDtypeStruct((NROW, 128), jnp.float32),
        mesh=mesh,
        scratch_types=[
            pltpu.VMEM((2, 64, 128), jnp.float32),
            pltpu.VMEM((2, 64, 128), jnp.float32),
            [pltpu.SemaphoreType.DMA] * 2,
            [pltpu.SemaphoreType.DMA] * 2,
        ],
        compiler_params=_params,
    )
    def ka(tableT, tailsc, scratch, tbuf, svout, fsems, wsems):
        w = lax.axis_index("s") * nc + lax.axis_index("c")

        def fire_fetch(k, b):
            ih = w + 32 * k
            for eh in range(8):
                pltpu.async_copy(
                    tableT.at[pl.ds(eh * 8, 8), pl.ds(ih * 128, 128)],
                    tbuf.at[b, pl.ds(eh * 8, 8)],
                    fsems[b],
                )

        def wait_fetch(b):
            for eh in range(8):
                pltpu.make_async_copy(
                    tableT.at[pl.ds(0, 8), pl.ds(0, 128)],
                    tbuf.at[b, pl.ds(eh * 8, 8)],
                    fsems[b],
                ).wait()

        def store(k, b):
            ih = w + 32 * k
            pltpu.async_copy(svout.at[b], scratch.at[pl.ds(ih * 64, 64)], wsems[b])

        def wait_store(b):
            pltpu.make_async_copy(
                svout.at[b], scratch.at[pl.ds(0, 64)], wsems[b]
            ).wait()

        fire_fetch(0, 0)
        fire_fetch(1, 1)

        def body(k, b, fetch_ahead, wait_wb):
            wait_fetch(b)
            if wait_wb:
                wait_store(b)
            _transpose_block(tbuf.at[b], svout.at[b], 64, 0)
            store(k, b)
            if fetch_ahead:
                fire_fetch(k + 2, b)

        def pair(i, carry):
            k = 2 + 2 * i
            body(k, 0, True, True)
            body(k + 1, 1, True, True)
            return carry

        body(0, 0, True, False)
        body(1, 1, True, False)
        lax.fori_loop(0, (n_main - 4) // 2, pair, 0)
        body(n_main - 2, 0, False, True)
        body(n_main - 1, 1, False, True)
        wait_store(0)
        wait_store(1)

        @pl.when(w >= 28)
        def _():
            ih = 7808 + (w - 28)
            for eh in range(8):
                pltpu.sync_copy(
                    tableT.at[pl.ds(eh * 8, 8), pl.ds(ih * 128, 128)],
                    tbuf.at[0, pl.ds(eh * 8, 8)],
                )
            _transpose_block(tbuf.at[0], svout.at[0], 64, 0)
            pltpu.sync_copy(svout.at[0], scratch.at[pl.ds(ih * 64, 64)])

        @pl.when(w == 27)
        def _():
            pltpu.sync_copy(tailsc, tbuf.at[0, pl.ds(0, 32)])
            pltpu.sync_copy(
                tbuf.at[0, pl.ds(0, 32)], scratch.at[pl.ds(NROW - 32, 32)]
            )

    return ka


@functools.cache
def _build_b(nc):
    mesh = plsc.VectorSubcoreMesh(core_axis_name="c", subcore_axis_name="s")
    n_tc = NTC // 32

    @functools.partial(
        pl.kernel,
        out_type=jax.ShapeDtypeStruct((OUTROWS, 128), jnp.float32),
        mesh=mesh,
        scratch_types=[
            pltpu.VMEM((2, 128), jnp.int32),
            pltpu.VMEM((2, 128), jnp.int32),
            pltpu.VMEM((2, 128), jnp.int32),
            pltpu.VMEM((2, 128, 128), jnp.float32),
            pltpu.VMEM((2, 64, 128), jnp.float32),
            [pltpu.SemaphoreType.DMA] * 2,
            [pltpu.SemaphoreType.DMA] * 2,
            [pltpu.SemaphoreType.DMA] * 2,
        ],
        compiler_params=_params,
    )
    def kb(xT, scratch, out2d, ivec, gvec, pvec, rows, outv, isems, gsems, osems):
        w = lax.axis_index("s") * nc + lax.axis_index("c")
        lanes = _iota16()

        def fire_idx(k, b):
            tc = w + 32 * k
            l = tc // 32
            bc = tc % 32
            pltpu.async_copy(
                xT.at[l, pl.ds(bc * 128, 128)], ivec.at[b], isems[b]
            )

        def wait_idx(b):
            pltpu.make_async_copy(
                xT.at[0, pl.ds(0, 128)], ivec.at[b], isems[b]
            ).wait()

        def prep_and_gather(b):
            for j16 in range(8):
                v = ivec[b, pl.ds(j16 * 16, 16)]
                gvec[b, pl.ds(j16 * 16, 16)] = lax.shift_right_logical(v, 1)
                pvec[b, pl.ds(j16 * 16, 16)] = lax.shift_left(
                    lax.bitwise_and(v, 1), 6
                )
            pltpu.async_copy(scratch.at[gvec.at[b]], rows.at[b], gsems[b])

        def wait_gather(b):
            pltpu.make_async_copy(
                scratch.at[gvec.at[b]], rows.at[b], gsems[b]
            ).wait()

        def transpose_tc(b):
            pvv = [pvec[b, pl.ds(j16 * 16, 16)] for j16 in range(8)]
            src = rows.at[b]
            dst = outv.at[b]

            @plsc.parallel_loop(0, 64, unroll=8)
            def col(e):
                for j16 in range(8):
                    g = plsc.load_gather(src, [lanes + j16 * 16, pvv[j16] + e])
                    dst[e, pl.ds(j16 * 16, 16)] = g

        def store_out(k, b):
            tc = w + 32 * k
            l = tc // 32
            bc = tc % 32
            row0 = l * 2048 + bc * 8
            for eh in range(8):
                pltpu.async_copy(
                    outv.at[b, pl.ds(eh * 8, 8)],
                    out2d.at[pl.ds(row0 + eh * 256, 8)],
                    osems[b],
                )

        def wait_store(b):
            for eh in range(8):
                pltpu.make_async_copy(
                    outv.at[b, pl.ds(0, 8)],
                    out2d.at[pl.ds(0, 8)],
                    osems[b],
                ).wait()

        fire_idx(0, 0)
        fire_idx(1, 1)
        wait_idx(0)
        prep_and_gather(0)

        def body(k, b, next_gather, fetch_ahead, wait_wb):
            nb = 1 - b
            wait_gather(b)
            if next_gather:
                wait_idx(nb)
                prep_and_gather(nb)
            if fetch_ahead:
                fire_idx(k + 2, b)
            if wait_wb:
                wait_store(b)
            transpose_tc(b)
            store_out(k, b)

        def pair(i, carry):
            k = 2 + 2 * i
            body(k, 0, True, True, True)
            body(k + 1, 1, True, True, True)
            return carry

        body(0, 0, True, True, False)
        body(1, 1, True, True, False)
        lax.fori_loop(0, (n_tc - 4) // 2, pair, 0)
        body(n_tc - 2, 0, True, False, True)
        body(n_tc - 1, 1, False, False, True)
        wait_store(0)
        wait_store(1)

    return kb


def kernel(x, table):
    info = plsc.get_sparse_core_info()
    nc = info.num_cores
    tableT = table.T
    tailsc = table[VOC - 64:].reshape(32, 128)
    xT = x.T
    scratch = _build_a(nc)(tableT, tailsc)
    out2d = _build_b(nc)(xT, scratch)
    out = out2d.reshape(NL, 8, NB // 128, 8, 128)
    out = out.transpose(2, 4, 0, 1, 3)
    return out.reshape(NB, NL, EMB)


# speedup vs baseline: 2.0270x; 1.0884x over previous
import functools

import jax
import jax.numpy as jnp
from jax import lax
from jax.experimental import pallas as pl
from jax.experimental.pallas import tpu as pltpu, tpu_sc as plsc

VOC = 1000000
EMB = 64
NB = 4096
NL = 200
NTOT = NB * NL
BLKV = 512
GRIDA = 977
NROWP = GRIDA * BLKV
NTC = (NB // 128) * NL
OUTROWS = NTOT * EMB // 128

_params = pltpu.CompilerParams(
    use_tc_tiling_on_sc=True, needs_layout_passes=False
)


def _iota16():
    return lax.iota(jnp.int32, 16)


def _ka_tc(a_ref, b_ref, o_ref):
    o_ref[...] = jnp.concatenate([a_ref[...].T, b_ref[...].T], axis=1)


def _stage_a(tableT):
    return pl.pallas_call(
        _ka_tc,
        grid=(GRIDA,),
        in_specs=[
            pl.BlockSpec((EMB, BLKV), lambda i: (0, i)),
            pl.BlockSpec((EMB, BLKV), lambda i: (0, i + GRIDA)),
        ],
        out_specs=pl.BlockSpec((BLKV, 128), lambda i: (i, 0)),
        out_shape=jax.ShapeDtypeStruct((NROWP, 128), jnp.float32),
        compiler_params=pltpu.CompilerParams(
            dimension_semantics=("parallel",)
        ),
    )(tableT, tableT)


@functools.cache
def _build_b(nc):
    mesh = plsc.VectorSubcoreMesh(core_axis_name="c", subcore_axis_name="s")
    n_tc = NTC // 32

    @functools.partial(
        pl.kernel,
        out_type=jax.ShapeDtypeStruct((OUTROWS, 128), jnp.float32),
        mesh=mesh,
        scratch_types=[
            pltpu.VMEM((2, 128), jnp.int32),
            pltpu.VMEM((2, 128), jnp.int32),
            pltpu.VMEM((2, 128), jnp.int32),
            pltpu.VMEM((2, 128, 128), jnp.float32),
            pltpu.VMEM((2, 64, 128), jnp.float32),
            [pltpu.SemaphoreType.DMA] * 2,
            [pltpu.SemaphoreType.DMA] * 2,
            [pltpu.SemaphoreType.DMA] * 2,
        ],
        compiler_params=_params,
    )
    def kb(xT, scratch, out2d, ivec, gvec, pvec, rows, outv, isems, gsems, osems):
        w = lax.axis_index("s") * nc + lax.axis_index("c")
        lanes = _iota16()

        def fire_idx(k, b):
            tc = w + 32 * k
            l = tc // 32
            bc = tc % 32
            pltpu.async_copy(
                xT.at[l, pl.ds(bc * 128, 128)], ivec.at[b], isems[b]
            )

        def wait_idx(b):
            pltpu.make_async_copy(
                xT.at[0, pl.ds(0, 128)], ivec.at[b], isems[b]
            ).wait()

        def prep_and_gather(b):
            for j16 in range(8):
                v = ivec[b, pl.ds(j16 * 16, 16)]
                d = lax.sub(v, jnp.int32(NROWP))
                m = lax.sub(jnp.int32(0), lax.shift_right_logical(d, 31))
                gvec[b, pl.ds(j16 * 16, 16)] = lax.bitwise_or(
                    lax.bitwise_and(v, m),
                    lax.bitwise_and(d, lax.bitwise_not(m)),
                )
                pvec[b, pl.ds(j16 * 16, 16)] = lax.bitwise_and(
                    lax.bitwise_not(m), jnp.int32(64)
                )
            pltpu.async_copy(scratch.at[gvec.at[b]], rows.at[b], gsems[b])

        def wait_gather(b):
            pltpu.make_async_copy(
                scratch.at[gvec.at[b]], rows.at[b], gsems[b]
            ).wait()

        def transpose_tc(b):
            pvv = [pvec[b, pl.ds(j16 * 16, 16)] for j16 in range(8)]
            src = rows.at[b]
            dst = outv.at[b]

            @plsc.parallel_loop(0, 64, unroll=8)
            def col(e):
                for j16 in range(8):
                    g = plsc.load_gather(src, [lanes + j16 * 16, pvv[j16] + e])
                    dst[e, pl.ds(j16 * 16, 16)] = g

        def store_out(k, b):
            tc = w + 32 * k
            l = tc // 32
            bc = tc % 32
            row0 = l * 2048 + bc * 8
            for eh in range(8):
                pltpu.async_copy(
                    outv.at[b, pl.ds(eh * 8, 8)],
                    out2d.at[pl.ds(row0 + eh * 256, 8)],
                    osems[b],
                )

        def wait_store(b):
            for eh in range(8):
                pltpu.make_async_copy(
                    outv.at[b, pl.ds(0, 8)],
                    out2d.at[pl.ds(0, 8)],
                    osems[b],
                ).wait()

        fire_idx(0, 0)
        fire_idx(1, 1)
        wait_idx(0)
        prep_and_gather(0)

        def body(k, b, next_gather, fetch_ahead, wait_wb):
            nb = 1 - b
            wait_gather(b)
            if next_gather:
                wait_idx(nb)
                prep_and_gather(nb)
            if fetch_ahead:
                fire_idx(k + 2, b)
            if wait_wb:
                wait_store(b)
            transpose_tc(b)
            store_out(k, b)

        def pair(i, carry):
            k = 2 + 2 * i
            body(k, 0, True, True, True)
            body(k + 1, 1, True, True, True)
            return carry

        body(0, 0, True, True, False)
        body(1, 1, True, True, False)
        lax.fori_loop(0, (n_tc - 4) // 2, pair, 0)
        body(n_tc - 2, 0, True, False, True)
        body(n_tc - 1, 1, False, False, True)
        wait_store(0)
        wait_store(1)

    return kb


def kernel(x, table):
    info = plsc.get_sparse_core_info()
    nc = info.num_cores
    tableT = table.T
    xT = x.T
    scratch = _stage_a(tableT)
    out2d = _build_b(nc)(xT, scratch)
    out = out2d.reshape(NL, 8, NB // 128, 8, 128)
    out = out.transpose(2, 4, 0, 1, 3)
    return out.reshape(NB, NL, EMB)
